# Initial kernel scaffold; baseline (speedup 1.0000x reference)
import jax
import jax.numpy as jnp
from jax.experimental import pallas as pl


def kernel(x, edge_index, batch_idx, W1, b1, W2, b2, W3, b3, Wh, bh):
    raise NotImplementedError("write your pallas kernel here")


# speedup vs baseline: 7.5645x; 7.5645x over previous
import functools

import jax
import jax.numpy as jnp
from jax import lax
from jax.experimental import pallas as pl
from jax.experimental.pallas import tpu as pltpu
from jax.experimental.pallas import tpu_sc as plsc

N = 10000
E = 320000
G = 128
C = 128

NC = 2
NS = 16
NW = NC * NS
EPW = E // NW
CHUNK = 128
NCH = -(-EPW // CHUNK)
EPW_PAD = NCH * CHUNK
AGG_ROWS = 10240
SLAB = AGG_ROWS // NS
DUMMY = N

BLK = 2000
NBLKS = N // BLK


def _mesh():
    return plsc.VectorSubcoreMesh(core_axis_name="c", subcore_axis_name="s")


def _sc_degree(dst_pad):
    @functools.partial(
        pl.kernel,
        out_type=jax.ShapeDtypeStruct((NC, AGG_ROWS, 16), jnp.float32),
        mesh=_mesh(),
        scratch_types=[
            pltpu.VMEM((NCH, CHUNK), jnp.int32),
            pltpu.VMEM((CHUNK, 16), jnp.float32),
            pltpu.VMEM_SHARED((AGG_ROWS, 16), jnp.float32),
        ],
    )
    def body(dst_hbm, out_hbm, didx, buf, hist):
        c = lax.axis_index("c")
        s = lax.axis_index("s")
        pltpu.sync_copy(dst_hbm.at[c, s], didx)

        z16 = jnp.zeros((16,), jnp.float32)

        def zrow(i, carry):
            buf[i, :] = z16
            return carry

        lax.fori_loop(0, CHUNK, zrow, 0)
        for k in range(SLAB // CHUNK):
            pltpu.sync_copy(buf, hist.at[pl.ds(s * SLAB + k * CHUNK, CHUNK)])

        e0 = jnp.where(lax.iota(jnp.int32, 16) == 0,
                       jnp.float32(1.0), jnp.float32(0.0))

        def orow(i, carry):
            buf[i, :] = e0
            return carry

        lax.fori_loop(0, CHUNK, orow, 0)
        plsc.subcore_barrier()

        def step(j, carry):
            pltpu.sync_copy(buf, hist.at[didx.at[j]], add=True)
            return carry

        lax.fori_loop(0, NCH, step, 0)
        plsc.subcore_barrier()
        pltpu.sync_copy(hist.at[pl.ds(s * SLAB, SLAB)],
                        out_hbm.at[c, pl.ds(s * SLAB, SLAB)])

    return body(dst_pad)


def _sc_aggregate(yn, src_pad, dst_pad):
    @functools.partial(
        pl.kernel,
        out_type=jax.ShapeDtypeStruct((NC, AGG_ROWS, C), jnp.float32),
        mesh=_mesh(),
        scratch_types=[
            pltpu.VMEM((NCH, CHUNK), jnp.int32),
            pltpu.VMEM((NCH, CHUNK), jnp.int32),
            pltpu.VMEM((CHUNK, C), jnp.float32),
            pltpu.VMEM_SHARED((AGG_ROWS, C), jnp.float32),
            pltpu.SemaphoreType.DMA,
        ],
    )
    def body(yn_hbm, src_hbm, dst_hbm, out_hbm, sidx, didx, rows, agg, sem):
        c = lax.axis_index("c")
        s = lax.axis_index("s")
        pltpu.sync_copy(src_hbm.at[c, s], sidx)
        pltpu.sync_copy(dst_hbm.at[c, s], didx)

        z16 = jnp.zeros((16,), jnp.float32)

        def zrow(i, carry):
            for k in range(C // 16):
                rows[i, pl.ds(k * 16, 16)] = z16
            return carry

        lax.fori_loop(0, CHUNK, zrow, 0)
        for k in range(SLAB // CHUNK):
            pltpu.sync_copy(rows, agg.at[pl.ds(s * SLAB + k * CHUNK, CHUNK)])
        plsc.subcore_barrier()

        def step(j, carry):
            pltpu.async_copy(yn_hbm.at[sidx.at[j]], rows, sem).wait()
            pltpu.sync_copy(rows, agg.at[didx.at[j]], add=True)
            return carry

        lax.fori_loop(0, NCH, step, 0)
        plsc.subcore_barrier()
        pltpu.sync_copy(agg.at[pl.ds(s * SLAB, SLAB)],
                        out_hbm.at[c, pl.ds(s * SLAB, SLAB)])

    return body(yn, src_pad, dst_pad)


def _tc_first(x, hist, W1):
    def body(x_ref, h0_ref, h1_ref, w_ref, yn_ref, norm_ref):
        d = h0_ref[0, :, 0:1] + h1_ref[0, :, 0:1] + 1.0
        nrm = lax.rsqrt(d)
        y = jnp.dot(x_ref[...], w_ref[...], preferred_element_type=jnp.float32)
        yn_ref[...] = y * nrm
        norm_ref[...] = nrm

    return pl.pallas_call(
        body,
        grid=(NBLKS,),
        in_specs=[
            pl.BlockSpec((BLK, C), lambda i: (i, 0)),
            pl.BlockSpec((1, BLK, 16), lambda i: (0, i, 0)),
            pl.BlockSpec((1, BLK, 16), lambda i: (1, i, 0)),
            pl.BlockSpec((C, C), lambda i: (0, 0)),
        ],
        out_specs=[
            pl.BlockSpec((BLK, C), lambda i: (i, 0)),
            pl.BlockSpec((BLK, 1), lambda i: (i, 0)),
        ],
        out_shape=[
            jax.ShapeDtypeStruct((N, C), jnp.float32),
            jax.ShapeDtypeStruct((N, 1), jnp.float32),
        ],
    )(x, hist, hist, W1)


def _tc_mid(p, yn, norm, b, W):
    def body(p0_ref, p1_ref, yn_ref, norm_ref, b_ref, w_ref, out_ref):
        nrm = norm_ref[...]
        h = jnp.maximum(
            (p0_ref[0] + p1_ref[0] + yn_ref[...]) * nrm + b_ref[...], 0.0)
        out_ref[...] = jnp.dot(
            h, w_ref[...], preferred_element_type=jnp.float32) * nrm

    return pl.pallas_call(
        body,
        grid=(NBLKS,),
        in_specs=[
            pl.BlockSpec((1, BLK, C), lambda i: (0, i, 0)),
            pl.BlockSpec((1, BLK, C), lambda i: (1, i, 0)),
            pl.BlockSpec((BLK, C), lambda i: (i, 0)),
            pl.BlockSpec((BLK, 1), lambda i: (i, 0)),
            pl.BlockSpec((1, C), lambda i: (0, 0)),
            pl.BlockSpec((C, C), lambda i: (0, 0)),
        ],
        out_specs=pl.BlockSpec((BLK, C), lambda i: (i, 0)),
        out_shape=jax.ShapeDtypeStruct((N, C), jnp.float32),
    )(p, p, yn, norm, b, W)


def _tc_head(p, yn, norm, b, batch3, Wh, bh):
    def body(p0_ref, p1_ref, yn_ref, norm_ref, b_ref, bi_ref, wh_ref, bh_ref,
             out_ref, pooled, counts):
        i = pl.program_id(0)

        @pl.when(i == 0)
        def _init():
            pooled[...] = jnp.zeros_like(pooled)
            counts[...] = jnp.zeros_like(counts)

        nrm = norm_ref[...]
        h = jnp.maximum(
            (p0_ref[0] + p1_ref[0] + yn_ref[...]) * nrm + b_ref[...], 0.0)
        bi = bi_ref[0, 0, :]
        oh = (bi[:, None] == lax.broadcasted_iota(jnp.int32, (BLK, G), 1)
              ).astype(jnp.float32)
        dn = (((0,), (0,)), ((), ()))
        pooled[...] += lax.dot_general(
            oh, h, dn, preferred_element_type=jnp.float32)
        counts[...] += lax.dot_general(
            oh, jnp.ones((BLK, 1), jnp.float32), dn,
            preferred_element_type=jnp.float32)

        @pl.when(i == NBLKS - 1)
        def _fin():
            mean = pooled[...] / jnp.maximum(counts[...], 1.0)
            out_ref[...] = jnp.dot(
                mean, wh_ref[...], preferred_element_type=jnp.float32
            ) + bh_ref[...]

    return pl.pallas_call(
        body,
        grid=(NBLKS,),
        in_specs=[
            pl.BlockSpec((1, BLK, C), lambda i: (0, i, 0)),
            pl.BlockSpec((1, BLK, C), lambda i: (1, i, 0)),
            pl.BlockSpec((BLK, C), lambda i: (i, 0)),
            pl.BlockSpec((BLK, 1), lambda i: (i, 0)),
            pl.BlockSpec((1, C), lambda i: (0, 0)),
            pl.BlockSpec((1, 1, BLK), lambda i: (i, 0, 0)),
            pl.BlockSpec((C, 1), lambda i: (0, 0)),
            pl.BlockSpec((1, 1), lambda i: (0, 0)),
        ],
        out_specs=pl.BlockSpec((G, 1), lambda i: (0, 0)),
        out_shape=jax.ShapeDtypeStruct((G, 1), jnp.float32),
        scratch_shapes=[
            pltpu.VMEM((G, C), jnp.float32),
            pltpu.VMEM((G, 1), jnp.float32),
        ],
    )(p, p, yn, norm, b, batch3, Wh, bh)


def kernel(x, edge_index, batch_idx, W1, b1, W2, b2, W3, b3, Wh, bh):
    src = edge_index[0].reshape(NW, EPW)
    dst = edge_index[1].reshape(NW, EPW)
    pad = EPW_PAD - EPW
    src_pad = jnp.pad(src, ((0, 0), (0, pad))).reshape(NC, NS, NCH, CHUNK)
    dst_pad = jnp.pad(dst, ((0, 0), (0, pad)),
                      constant_values=DUMMY).reshape(NC, NS, NCH, CHUNK)
    batch3 = batch_idx.reshape(NBLKS, 1, BLK)

    hist = _sc_degree(dst_pad)
    yn1, norm = _tc_first(x, hist, W1)
    p1 = _sc_aggregate(yn1, src_pad, dst_pad)
    yn2 = _tc_mid(p1, yn1, norm, b1.reshape(1, C), W2)
    p2 = _sc_aggregate(yn2, src_pad, dst_pad)
    yn3 = _tc_mid(p2, yn2, norm, b2.reshape(1, C), W3)
    p3 = _sc_aggregate(yn3, src_pad, dst_pad)
    out = _tc_head(p3, yn3, norm, b3.reshape(1, C), batch3,
                   Wh, bh.reshape(1, 1))
    return jnp.squeeze(out, axis=-1)
